# Initial kernel scaffold; baseline (speedup 1.0000x reference)
import jax
import jax.numpy as jnp
from jax.experimental import pallas as pl


def kernel(inputs, word_table, pos_table):
    raise NotImplementedError("write your pallas kernel here")


# speedup vs baseline: 1.6692x; 1.6692x over previous
import functools

import jax
import jax.numpy as jnp
from jax import lax
from jax.experimental import pallas as pl
from jax.experimental.pallas import tpu as pltpu
from jax.experimental.pallas import tpu_sc as plsc

VOCAB = 10000
D = 100
SEQ = 200
BATCH = 4096
ROWS = BATCH * SEQ

NC = 2
NS = 16
NW = NC * NS
ROWS_PER_W = ROWS // NW
CHUNK = SEQ
NCHUNK = ROWS_PER_W // CHUNK
SUB = 100
SUB_G = 104
DM = 96
DT = 16
TOFF = D - DT


def _body(main_hbm, tail_hbm, idx_hbm, posm_hbm, post_hbm, out_hbm,
          idx_v, posm_v, post_v, main_v, tail_v, rows_v, sem):
    wid = lax.axis_index("c") * NS + lax.axis_index("s")
    pltpu.sync_copy(idx_hbm.at[wid], idx_v)
    pltpu.sync_copy(posm_hbm, posm_v)
    pltpu.sync_copy(post_hbm, post_v)
    base = wid * ROWS_PER_W

    def chunk_body(g, _):
        j0 = g * 2
        cps = []
        for i in range(2):
            cps.append(pltpu.async_copy(
                main_hbm.at[idx_v.at[j0 + i]],
                main_v.at[pl.ds(i * SUB, SUB_G)], sem))
            cps.append(pltpu.async_copy(
                tail_hbm.at[idx_v.at[j0 + i]],
                tail_v.at[pl.ds(i * SUB, SUB_G)], sem))
        for cp in cps:
            cp.wait()

        def row_body(r, carry):
            for j in range(6):
                rows_v[r, pl.ds(j * 16, 16)] = (
                    main_v[r, pl.ds(j * 16, 16)]
                    + posm_v[r, pl.ds(j * 16, 16)])
            rows_v[r, pl.ds(TOFF, 16)] = tail_v[r, :] + post_v[r, :]
            return carry

        lax.fori_loop(0, CHUNK, row_body, 0)
        pltpu.sync_copy(rows_v,
                        out_hbm.at[pl.ds(base + g * CHUNK, CHUNK)])
        return 0

    lax.fori_loop(0, NCHUNK, chunk_body, 0)


@functools.partial(
    pl.kernel,
    out_type=jax.ShapeDtypeStruct((ROWS, D), jnp.float32),
    mesh=plsc.VectorSubcoreMesh(core_axis_name="c", subcore_axis_name="s"),
    scratch_types=[
        pltpu.VMEM((2 * NCHUNK, SUB_G), jnp.int32),
        pltpu.VMEM((SEQ, DM), jnp.float32),
        pltpu.VMEM((SEQ, DT), jnp.float32),
        pltpu.VMEM((CHUNK + (SUB_G - SUB), DM), jnp.float32),
        pltpu.VMEM((CHUNK + (SUB_G - SUB), DT), jnp.float32),
        pltpu.VMEM((CHUNK, D), jnp.float32),
        pltpu.SemaphoreType.DMA,
    ],
    compiler_params=pltpu.CompilerParams(use_tc_tiling_on_sc=False),
)
def _embed_kernel(main_hbm, tail_hbm, idx_hbm, posm_hbm, post_hbm, out_hbm,
                  idx_v, posm_v, post_v, main_v, tail_v, rows_v, sem):
    _body(main_hbm, tail_hbm, idx_hbm, posm_hbm, post_hbm, out_hbm,
          idx_v, posm_v, post_v, main_v, tail_v, rows_v, sem)


def kernel(inputs, word_table, pos_table):
    idx = inputs.reshape(ROWS // SUB, SUB).astype(jnp.int32)
    idx = jnp.concatenate([idx, jnp.roll(idx, -1, axis=0)[:, :4]], axis=1)
    idx = idx.reshape(NW, 2 * NCHUNK, SUB_G)
    out = _embed_kernel(word_table[:, :DM], word_table[:, TOFF:], idx,
                        pos_table[:, :DM], pos_table[:, TOFF:])
    return out.reshape(BATCH, SEQ, D)


# speedup vs baseline: 1.9842x; 1.1887x over previous
import functools

import jax
import jax.numpy as jnp
from jax import lax
from jax.experimental import pallas as pl
from jax.experimental.pallas import tpu as pltpu
from jax.experimental.pallas import tpu_sc as plsc

VOCAB = 10000
D = 100
SEQ = 200
BATCH = 4096
ROWS = BATCH * SEQ

NC = 2
NS = 16
NW = NC * NS
ROWS_PER_W = ROWS // NW
CHUNK = 100
NCHUNK = ROWS_PER_W // CHUNK
SUB_G = 104
DM = 96
DT = 16
TOFF = D - DT


def _body(main_hbm, tail_hbm, idx_hbm, posm_hbm, post_hbm, out_hbm,
          idx_v, posm_v, post_v, main_v, tail_v, rows_v,
          semg0, semg1, semo0, semo1):
    wid = lax.axis_index("c") * NS + lax.axis_index("s")
    pltpu.sync_copy(idx_hbm.at[wid], idx_v)
    pltpu.sync_copy(posm_hbm, posm_v)
    pltpu.sync_copy(post_hbm, post_v)
    base = wid * ROWS_PER_W
    semg = (semg0, semg1)
    semo = (semo0, semo1)

    def fire_gathers(g, b):
        pltpu.async_copy(main_hbm.at[idx_v.at[g]], main_v.at[b], semg[b])
        pltpu.async_copy(tail_hbm.at[idx_v.at[g]], tail_v.at[b], semg[b])

    def wait_gathers(b):
        pltpu.make_async_copy(
            main_hbm.at[idx_v.at[0]], main_v.at[b], semg[b]).wait()
        pltpu.make_async_copy(
            tail_hbm.at[idx_v.at[0]], tail_v.at[b], semg[b]).wait()

    def wait_out(b):
        pltpu.make_async_copy(
            rows_v.at[b], out_hbm.at[pl.ds(0, CHUNK)], semo[b]).wait()

    def merge(b):
        p0 = b * CHUNK

        def row_body(r, carry):
            for j in range(6):
                rows_v[b, r, pl.ds(j * 16, 16)] = (
                    main_v[b, r, pl.ds(j * 16, 16)]
                    + posm_v[p0 + r, pl.ds(j * 16, 16)])
            rows_v[b, r, pl.ds(TOFF, 16)] = (
                tail_v[b, r, :] + post_v[p0 + r, :])
            return carry

        lax.fori_loop(0, CHUNK, row_body, 0, unroll=2)

    fire_gathers(0, 0)

    def pair_body(gg, carry):
        for b in range(2):
            g = gg * 2 + b

            @pl.when(g + 1 < NCHUNK)
            def _():
                fire_gathers(g + 1, 1 - b)

            wait_gathers(b)

            @pl.when(gg >= 1)
            def _():
                wait_out(b)

            merge(b)
            pltpu.async_copy(
                rows_v.at[b], out_hbm.at[pl.ds(base + g * CHUNK, CHUNK)],
                semo[b])
        return carry

    lax.fori_loop(0, NCHUNK // 2, pair_body, 0)
    wait_out(0)
    wait_out(1)


@functools.partial(
    pl.kernel,
    out_type=jax.ShapeDtypeStruct((ROWS, D), jnp.float32),
    mesh=plsc.VectorSubcoreMesh(core_axis_name="c", subcore_axis_name="s"),
    scratch_types=[
        pltpu.VMEM((NCHUNK, SUB_G), jnp.int32),
        pltpu.VMEM((SEQ, DM), jnp.float32),
        pltpu.VMEM((SEQ, DT), jnp.float32),
        pltpu.VMEM((2, SUB_G, DM), jnp.float32),
        pltpu.VMEM((2, SUB_G, DT), jnp.float32),
        pltpu.VMEM((2, CHUNK, D), jnp.float32),
        pltpu.SemaphoreType.DMA,
        pltpu.SemaphoreType.DMA,
        pltpu.SemaphoreType.DMA,
        pltpu.SemaphoreType.DMA,
    ],
    compiler_params=pltpu.CompilerParams(use_tc_tiling_on_sc=False),
)
def _embed_kernel(main_hbm, tail_hbm, idx_hbm, posm_hbm, post_hbm, out_hbm,
                  idx_v, posm_v, post_v, main_v, tail_v, rows_v,
                  semg0, semg1, semo0, semo1):
    _body(main_hbm, tail_hbm, idx_hbm, posm_hbm, post_hbm, out_hbm,
          idx_v, posm_v, post_v, main_v, tail_v, rows_v,
          semg0, semg1, semo0, semo1)


def kernel(inputs, word_table, pos_table):
    idx = inputs.reshape(ROWS // CHUNK, CHUNK).astype(jnp.int32)
    idx = jnp.concatenate([idx, jnp.roll(idx, -1, axis=0)[:, :4]], axis=1)
    idx = idx.reshape(NW, NCHUNK, SUB_G)
    out = _embed_kernel(word_table[:, :DM], word_table[:, TOFF:], idx,
                        pos_table[:, :DM], pos_table[:, TOFF:])
    return out.reshape(BATCH, SEQ, D)


# speedup vs baseline: 1.9866x; 1.0012x over previous
import functools

import jax
import jax.numpy as jnp
from jax import lax
from jax.experimental import pallas as pl
from jax.experimental.pallas import tpu as pltpu
from jax.experimental.pallas import tpu_sc as plsc

VOCAB = 10000
D = 100
SEQ = 200
BATCH = 4096
ROWS = BATCH * SEQ

NC = 2
NS = 16
NW = NC * NS
ROWS_PER_W = ROWS // NW
CHUNK = 100
NCHUNK = ROWS_PER_W // CHUNK
SUB_G = 104
DM = 96
DT = 16
TOFF = D - DT


def _body(main_hbm, tail_hbm, idx_hbm, posm_hbm, post_hbm, out_hbm,
          idx_v, posm_v, post_v, main_v, tail_v, rows_v,
          semg0, semg1, semo0, semo1):
    wid = lax.axis_index("c") * NS + lax.axis_index("s")
    pltpu.sync_copy(idx_hbm.at[wid], idx_v)
    pltpu.sync_copy(posm_hbm, posm_v)
    pltpu.sync_copy(post_hbm, post_v)
    bbase = wid * (ROWS_PER_W // SEQ)
    semg = (semg0, semg1)
    semo = (semo0, semo1)

    def fire_gathers(g, b):
        pltpu.async_copy(main_hbm.at[idx_v.at[g]], main_v.at[b], semg[b])
        pltpu.async_copy(tail_hbm.at[idx_v.at[g]], tail_v.at[b], semg[b])

    def wait_gathers(b):
        pltpu.make_async_copy(
            main_hbm.at[idx_v.at[0]], main_v.at[b], semg[b]).wait()
        pltpu.make_async_copy(
            tail_hbm.at[idx_v.at[0]], tail_v.at[b], semg[b]).wait()

    def wait_out(b):
        pltpu.make_async_copy(
            rows_v.at[b], out_hbm.at[0, pl.ds(0, CHUNK)], semo[b]).wait()

    def merge(b):
        p0 = b * CHUNK

        def row_body(r, carry):
            for j in range(6):
                rows_v[b, r, pl.ds(j * 16, 16)] = (
                    main_v[b, r, pl.ds(j * 16, 16)]
                    + posm_v[p0 + r, pl.ds(j * 16, 16)])
            rows_v[b, r, pl.ds(TOFF, 16)] = (
                tail_v[b, r, :] + post_v[p0 + r, :])
            return carry

        lax.fori_loop(0, CHUNK, row_body, 0, unroll=2)

    fire_gathers(0, 0)

    def pair_body(gg, carry):
        for b in range(2):
            g = gg * 2 + b

            @pl.when(g + 1 < NCHUNK)
            def _():
                fire_gathers(g + 1, 1 - b)

            wait_gathers(b)

            @pl.when(gg >= 1)
            def _():
                wait_out(b)

            merge(b)
            pltpu.async_copy(
                rows_v.at[b],
                out_hbm.at[bbase + gg, pl.ds(b * CHUNK, CHUNK)],
                semo[b])
        return carry

    lax.fori_loop(0, NCHUNK // 2, pair_body, 0)
    wait_out(0)
    wait_out(1)


@functools.partial(
    pl.kernel,
    out_type=jax.ShapeDtypeStruct((BATCH, SEQ, D), jnp.float32),
    mesh=plsc.VectorSubcoreMesh(core_axis_name="c", subcore_axis_name="s"),
    scratch_types=[
        pltpu.VMEM((NCHUNK, SUB_G), jnp.int32),
        pltpu.VMEM((SEQ, DM), jnp.float32),
        pltpu.VMEM((SEQ, DT), jnp.float32),
        pltpu.VMEM((2, SUB_G, DM), jnp.float32),
        pltpu.VMEM((2, SUB_G, DT), jnp.float32),
        pltpu.VMEM((2, CHUNK, D), jnp.float32),
        pltpu.SemaphoreType.DMA,
        pltpu.SemaphoreType.DMA,
        pltpu.SemaphoreType.DMA,
        pltpu.SemaphoreType.DMA,
    ],
    compiler_params=pltpu.CompilerParams(use_tc_tiling_on_sc=False),
)
def _embed_kernel(main_hbm, tail_hbm, idx_hbm, posm_hbm, post_hbm, out_hbm,
                  idx_v, posm_v, post_v, main_v, tail_v, rows_v,
                  semg0, semg1, semo0, semo1):
    _body(main_hbm, tail_hbm, idx_hbm, posm_hbm, post_hbm, out_hbm,
          idx_v, posm_v, post_v, main_v, tail_v, rows_v,
          semg0, semg1, semo0, semo1)


def kernel(inputs, word_table, pos_table):
    idx = inputs.reshape(ROWS // CHUNK, CHUNK).astype(jnp.int32)
    idx = jnp.concatenate([idx, jnp.roll(idx, -1, axis=0)[:, :4]], axis=1)
    idx = idx.reshape(NW, NCHUNK, SUB_G)
    return _embed_kernel(word_table[:, :DM], word_table[:, TOFF:], idx,
                         pos_table[:, :DM], pos_table[:, TOFF:])


# speedup vs baseline: 1.9967x; 1.0051x over previous
import functools

import jax
import jax.numpy as jnp
from jax import lax
from jax.experimental import pallas as pl
from jax.experimental.pallas import tpu as pltpu
from jax.experimental.pallas import tpu_sc as plsc

VOCAB = 10000
D = 100
SEQ = 200
BATCH = 4096
ROWS = BATCH * SEQ

NC = 2
NS = 16
NW = NC * NS
ROWS_PER_W = ROWS // NW
CHUNK = 100
NCHUNK = ROWS_PER_W // CHUNK
SUB_G = 104
DP = 112
DT = 16
TOFF = D - DT


def _body(word_hbm, idx_hbm, posm_hbm, post_hbm, out_hbm,
          idx_v, posm_v, post_v, main_v, rows_v,
          semg0, semg1, semo0, semo1):
    wid = lax.axis_index("c") * NS + lax.axis_index("s")
    pltpu.sync_copy(idx_hbm.at[wid], idx_v)
    pltpu.sync_copy(posm_hbm, posm_v)
    pltpu.sync_copy(post_hbm, post_v)
    bbase = wid * (ROWS_PER_W // SEQ)
    semg = (semg0, semg1)
    semo = (semo0, semo1)

    def fire_gathers(g, b):
        pltpu.async_copy(word_hbm.at[idx_v.at[g]], main_v.at[b], semg[b])

    def wait_gathers(b):
        pltpu.make_async_copy(
            word_hbm.at[idx_v.at[0]], main_v.at[b], semg[b]).wait()

    def wait_out(b):
        pltpu.make_async_copy(
            rows_v.at[b], out_hbm.at[0, pl.ds(0, CHUNK)], semo[b]).wait()

    def merge(b):
        p0 = b * CHUNK

        def row_body(r, carry):
            for j in range(6):
                rows_v[b, r, pl.ds(j * 16, 16)] = (
                    main_v[b, r, pl.ds(j * 16, 16)]
                    + posm_v[p0 + r, pl.ds(j * 16, 16)])
            rows_v[b, r, pl.ds(TOFF, 16)] = (
                main_v[b, r, pl.ds(TOFF, 16)] + post_v[p0 + r, :])
            return carry

        lax.fori_loop(0, CHUNK, row_body, 0, unroll=4)

    fire_gathers(0, 0)

    def pair_body(gg, carry):
        for b in range(2):
            g = gg * 2 + b

            @pl.when(g + 1 < NCHUNK)
            def _():
                fire_gathers(g + 1, 1 - b)

            wait_gathers(b)

            @pl.when(gg >= 1)
            def _():
                wait_out(b)

            merge(b)
            pltpu.async_copy(
                rows_v.at[b],
                out_hbm.at[bbase + gg, pl.ds(b * CHUNK, CHUNK)],
                semo[b])
        return carry

    lax.fori_loop(0, NCHUNK // 2, pair_body, 0)
    wait_out(0)
    wait_out(1)


@functools.partial(
    pl.kernel,
    out_type=jax.ShapeDtypeStruct((BATCH, SEQ, D), jnp.float32),
    mesh=plsc.VectorSubcoreMesh(core_axis_name="c", subcore_axis_name="s"),
    scratch_types=[
        pltpu.VMEM((NCHUNK, SUB_G), jnp.int32),
        pltpu.VMEM((SEQ, DP), jnp.float32),
        pltpu.VMEM((SEQ, DT), jnp.float32),
        pltpu.VMEM((2, SUB_G, DP), jnp.float32),
        pltpu.VMEM((2, CHUNK, D), jnp.float32),
        pltpu.SemaphoreType.DMA,
        pltpu.SemaphoreType.DMA,
        pltpu.SemaphoreType.DMA,
        pltpu.SemaphoreType.DMA,
    ],
    compiler_params=pltpu.CompilerParams(use_tc_tiling_on_sc=False),
)
def _embed_kernel(word_hbm, idx_hbm, posm_hbm, post_hbm, out_hbm,
                  idx_v, posm_v, post_v, main_v, rows_v,
                  semg0, semg1, semo0, semo1):
    _body(word_hbm, idx_hbm, posm_hbm, post_hbm, out_hbm,
          idx_v, posm_v, post_v, main_v, rows_v,
          semg0, semg1, semo0, semo1)


def kernel(inputs, word_table, pos_table):
    idx = inputs.reshape(ROWS // CHUNK, CHUNK).astype(jnp.int32)
    idx = jnp.concatenate([idx, jnp.roll(idx, -1, axis=0)[:, :4]], axis=1)
    idx = idx.reshape(NW, NCHUNK, SUB_G)
    word_pad = jnp.pad(word_table, ((0, 0), (0, DP - D)))
    pos_pad = jnp.pad(pos_table, ((0, 0), (0, DP - D)))
    return _embed_kernel(word_pad, idx, pos_pad, pos_table[:, TOFF:])
